# baseline (speedup 1.0000x reference)
import functools

import jax
import jax.numpy as jnp
from jax import lax
from jax.experimental import pallas as pl
from jax.experimental.pallas import tpu as pltpu
from jax.experimental.pallas import tpu_sc as plsc

VOCAB_SIZE = 100000
EMB_DIM = 64
BATCH = 4096
CTX = 20

NUM_CORES = 2
NUM_SUBCORES = 16
NUM_WORKERS = NUM_CORES * NUM_SUBCORES
B_PER_W = BATCH // NUM_WORKERS
CHUNK = 64
NCHUNK = B_PER_W // CHUNK
LANES = 16


def _make_pool_kernel():
    mesh = plsc.VectorSubcoreMesh(core_axis_name="c", subcore_axis_name="s")

    @functools.partial(
        pl.kernel,
        mesh=mesh,
        out_type=jax.ShapeDtypeStruct((BATCH, EMB_DIM), jnp.float32),
        scratch_types=[
            pltpu.VMEM((CHUNK * CTX,), jnp.int32),
            pltpu.VMEM((CHUNK * CTX, EMB_DIM), jnp.float32),
            pltpu.VMEM((CHUNK, EMB_DIM), jnp.float32),
            pltpu.SemaphoreType.DMA,
        ],
        compiler_params=pltpu.CompilerParams(use_tc_tiling_on_sc=False),
    )
    def pool_kernel(idx_hbm, table_hbm, out_hbm, idx_v, rows_v, pooled_v, sem):
        wid = lax.axis_index("s") * NUM_CORES + lax.axis_index("c")
        for c in range(NCHUNK):
            base_row = wid * B_PER_W + c * CHUNK
            pltpu.sync_copy(idx_hbm.at[pl.ds(base_row * CTX, CHUNK * CTX)], idx_v)
            pltpu.async_copy(table_hbm.at[idx_v], rows_v, sem).wait()

            def body(i, _):
                b0 = i * CTX
                for d in range(EMB_DIM // LANES):
                    sl = pl.ds(d * LANES, LANES)
                    acc = rows_v[b0, sl]
                    for l in range(1, CTX):
                        acc = acc + rows_v[b0 + l, sl]
                    pooled_v[i, sl] = acc * (1.0 / CTX)
                return 0

            lax.fori_loop(0, CHUNK, body, 0)
            pltpu.sync_copy(pooled_v, out_hbm.at[pl.ds(base_row, CHUNK)])

    return pool_kernel


_pool = _make_pool_kernel()

BM = 4096
BN = 512

_LOG2E = 1.4426950408889634
_LN2 = 0.6931471805599453


def _mm_body(w_ref, p_ref, b_ref, o_ref):
    x = lax.dot_general(
        w_ref[...], p_ref[...],
        (((0,), (1,)), ((), ())),
        preferred_element_type=jnp.float32,
    )
    x = x + b_ref[...][:, None]
    z = jnp.exp2(jnp.abs(x) * (-_LOG2E))
    o_ref[...] = jnp.minimum(x, 0.0) - jnp.log(1.0 + z)


def _project(pooled, w_t, lin_b):
    n_tiles = pl.cdiv(VOCAB_SIZE, BN)
    m_tiles = BATCH // BM
    return pl.pallas_call(
        _mm_body,
        grid=(n_tiles, m_tiles),
        in_specs=[
            pl.BlockSpec((EMB_DIM, BN), lambda n, m: (0, n)),
            pl.BlockSpec((BM, EMB_DIM), lambda n, m: (m, 0)),
            pl.BlockSpec((BN,), lambda n, m: (n,)),
        ],
        out_specs=pl.BlockSpec((BN, BM), lambda n, m: (n, m)),
        out_shape=jax.ShapeDtypeStruct((VOCAB_SIZE, BATCH), jnp.float32),
        compiler_params=pltpu.CompilerParams(
            dimension_semantics=("parallel", "parallel"),
        ),
    )(w_t, pooled, lin_b)


def kernel(inputs, emb_table, lin_w, lin_b):
    idx_flat = inputs.reshape(-1).astype(jnp.int32)
    pooled = _pool(idx_flat, emb_table)
    return _project(pooled, lin_w.T, lin_b).T


# speedup vs baseline: 1.0721x; 1.0721x over previous
import functools

import jax
import jax.numpy as jnp
from jax import lax
from jax.experimental import pallas as pl
from jax.experimental.pallas import tpu as pltpu
from jax.experimental.pallas import tpu_sc as plsc

VOCAB_SIZE = 100000
EMB_DIM = 64
BATCH = 4096
CTX = 20

NUM_CORES = 2
NUM_SUBCORES = 16
NUM_WORKERS = NUM_CORES * NUM_SUBCORES
B_PER_W = BATCH // NUM_WORKERS
CHUNK = 64
NCHUNK = B_PER_W // CHUNK
LANES = 16


def _make_pool_kernel():
    mesh = plsc.VectorSubcoreMesh(core_axis_name="c", subcore_axis_name="s")

    @functools.partial(
        pl.kernel,
        mesh=mesh,
        out_type=jax.ShapeDtypeStruct((BATCH, EMB_DIM), jnp.float32),
        scratch_types=[
            pltpu.VMEM((CHUNK * CTX,), jnp.int32),
            pltpu.VMEM((CHUNK * CTX, EMB_DIM), jnp.float32),
            pltpu.VMEM((CHUNK, EMB_DIM), jnp.float32),
            pltpu.SemaphoreType.DMA,
        ],
        compiler_params=pltpu.CompilerParams(use_tc_tiling_on_sc=False),
    )
    def pool_kernel(idx_hbm, table_hbm, out_hbm, idx_v, rows_v, pooled_v, sem):
        wid = lax.axis_index("s") * NUM_CORES + lax.axis_index("c")
        for c in range(NCHUNK):
            base_row = wid * B_PER_W + c * CHUNK
            pltpu.sync_copy(idx_hbm.at[pl.ds(base_row * CTX, CHUNK * CTX)], idx_v)
            pltpu.async_copy(table_hbm.at[idx_v], rows_v, sem).wait()

            def body(i, _):
                b0 = i * CTX
                for d in range(EMB_DIM // LANES):
                    sl = pl.ds(d * LANES, LANES)
                    acc = rows_v[b0, sl]
                    for l in range(1, CTX):
                        acc = acc + rows_v[b0 + l, sl]
                    pooled_v[i, sl] = acc * (1.0 / CTX)
                return 0

            lax.fori_loop(0, CHUNK, body, 0)
            pltpu.sync_copy(pooled_v, out_hbm.at[pl.ds(base_row, CHUNK)])

    return pool_kernel


_pool = _make_pool_kernel()

BM = 4096
BN = 1024

_LOG2E = 1.4426950408889634
_LN2 = 0.6931471805599453


def _mm_body(w_ref, p_ref, b_ref, o_ref):
    x = lax.dot_general(
        w_ref[...], p_ref[...],
        (((0,), (1,)), ((), ())),
        preferred_element_type=jnp.float32,
    )
    x = x + b_ref[...][:, None]
    z = jnp.exp2(jnp.abs(x) * (-_LOG2E))
    o_ref[...] = jnp.minimum(x, 0.0) - jnp.log(1.0 + z)


def _project(pooled, w_t, lin_b):
    n_tiles = pl.cdiv(VOCAB_SIZE, BN)
    m_tiles = BATCH // BM
    return pl.pallas_call(
        _mm_body,
        grid=(n_tiles, m_tiles),
        in_specs=[
            pl.BlockSpec((EMB_DIM, BN), lambda n, m: (0, n)),
            pl.BlockSpec((BM, EMB_DIM), lambda n, m: (m, 0)),
            pl.BlockSpec((BN,), lambda n, m: (n,)),
        ],
        out_specs=pl.BlockSpec((BN, BM), lambda n, m: (n, m)),
        out_shape=jax.ShapeDtypeStruct((VOCAB_SIZE, BATCH), jnp.float32),
        compiler_params=pltpu.CompilerParams(
            dimension_semantics=("parallel", "parallel"),
        ),
    )(w_t, pooled, lin_b)


def kernel(inputs, emb_table, lin_w, lin_b):
    idx_flat = inputs.reshape(-1).astype(jnp.int32)
    pooled = _pool(idx_flat, emb_table)
    return _project(pooled, lin_w.T, lin_b).T
